# Initial kernel scaffold; baseline (speedup 1.0000x reference)
import jax
import jax.numpy as jnp
from jax.experimental import pallas as pl


def kernel(x, mapped_tokens, Wg, bg, Wi, bi, Wo, bo):
    raise NotImplementedError("write your pallas kernel here")


# speedup vs baseline: 3.4986x; 3.4986x over previous
import functools

import jax
import jax.numpy as jnp
from jax import lax
from jax.experimental import pallas as pl
from jax.experimental.pallas import tpu as pltpu
from jax.experimental.pallas import tpu_sc as plsc

DIM = 768
HID = DIM * 4
E = 16
T = 2048
TILE = 128
NT = T // TILE
W = NT + E - 1


def _make_sc_row_gather(n_rows, dim):
    info = plsc.get_sparse_core_info()
    nc, ns = info.num_cores, info.num_subcores
    nw = nc * ns
    assert n_rows % nw == 0
    per_w = n_rows // nw
    mesh = plsc.VectorSubcoreMesh(core_axis_name="c", subcore_axis_name="s")

    @functools.partial(
        pl.kernel,
        mesh=mesh,
        out_type=jax.ShapeDtypeStruct((n_rows, dim), jnp.float32),
        scratch_types=[
            pltpu.VMEM((per_w,), jnp.int32),
            pltpu.VMEM((per_w, dim), jnp.float32),
            pltpu.SemaphoreType.DMA,
        ],
    )
    def gather_k(table_hbm, idx_hbm, out_hbm, idx_v, rows_v, sem):
        wid = lax.axis_index("s") * nc + lax.axis_index("c")
        base = wid * per_w
        pltpu.sync_copy(idx_hbm.at[pl.ds(base, per_w)], idx_v)
        pltpu.async_copy(table_hbm.at[idx_v], rows_v, sem).wait()
        pltpu.sync_copy(rows_v, out_hbm.at[pl.ds(base, per_w)])

    return gather_k


def _ffn_body(tid_ref, eid_ref, lo_ref, hi_ref,
              x_ref, wg_ref, bg_ref, wi_ref, bi_ref, wo_ref, bo_ref,
              out_ref):
    w = pl.program_id(0)
    tid = tid_ref[w]
    prev_tid = tid_ref[jnp.maximum(w - 1, 0)]
    first_visit = jnp.logical_or(w == 0, tid != prev_tid)

    @pl.when(first_visit)
    def _init():
        out_ref[...] = jnp.zeros_like(out_ref)

    lo = lo_ref[w]
    hi = hi_ref[w]

    @pl.when(lo < hi)
    def _compute():
        xb = x_ref[...]
        g = jnp.dot(xb, wg_ref[0], preferred_element_type=jnp.float32)
        g = g + bg_ref[0]
        i_ = jnp.dot(xb, wi_ref[0], preferred_element_type=jnp.float32)
        i_ = i_ + bi_ref[0]
        h = (g * jax.nn.sigmoid(g)) * i_
        o = jnp.dot(h, wo_ref[0], preferred_element_type=jnp.float32)
        o = o + bo_ref[0]
        rows = tid * TILE + lax.broadcasted_iota(jnp.int32, (TILE, 1), 0)
        mask = jnp.logical_and(rows >= lo, rows < hi)
        out_ref[...] += jnp.where(mask, o, 0.0)


def _grouped_ffn(tile_w, exp_w, lo_w, hi_w, x_sorted, Wg, bg, Wi, bi, Wo, bo):
    grid_spec = pltpu.PrefetchScalarGridSpec(
        num_scalar_prefetch=4,
        grid=(W,),
        in_specs=[
            pl.BlockSpec((TILE, DIM), lambda w, tid, eid, lo, hi: (tid[w], 0)),
            pl.BlockSpec((1, DIM, HID), lambda w, tid, eid, lo, hi: (eid[w], 0, 0)),
            pl.BlockSpec((1, 1, HID), lambda w, tid, eid, lo, hi: (eid[w], 0, 0)),
            pl.BlockSpec((1, DIM, HID), lambda w, tid, eid, lo, hi: (eid[w], 0, 0)),
            pl.BlockSpec((1, 1, HID), lambda w, tid, eid, lo, hi: (eid[w], 0, 0)),
            pl.BlockSpec((1, HID, DIM), lambda w, tid, eid, lo, hi: (eid[w], 0, 0)),
            pl.BlockSpec((1, 1, DIM), lambda w, tid, eid, lo, hi: (eid[w], 0, 0)),
        ],
        out_specs=pl.BlockSpec((TILE, DIM), lambda w, tid, eid, lo, hi: (tid[w], 0)),
    )
    return pl.pallas_call(
        _ffn_body,
        grid_spec=grid_spec,
        out_shape=jax.ShapeDtypeStruct((T, DIM), jnp.float32),
    )(tile_w, exp_w, lo_w, hi_w, x_sorted,
      Wg, bg.reshape(E, 1, HID), Wi, bi.reshape(E, 1, HID),
      Wo, bo.reshape(E, 1, DIM))


def _worklist(mapped_tokens):
    mt = mapped_tokens.astype(jnp.int32)
    counts = jnp.bincount(mt, length=E)
    off = jnp.concatenate([jnp.zeros((1,), jnp.int32),
                           jnp.cumsum(counts).astype(jnp.int32)])
    t_base = jnp.arange(NT, dtype=jnp.int32)[:, None] * TILE
    lo = jnp.maximum(off[:-1][None, :], t_base)
    hi = jnp.minimum(off[1:][None, :], t_base + TILE)
    valid = (hi > lo).ravel()
    seq = jnp.arange(NT * E, dtype=jnp.int32)
    order_key = jnp.where(valid, seq, seq + NT * E)
    sel = jnp.argsort(order_key)[:W]
    tile_f = (sel // E).astype(jnp.int32)
    exp_f = (sel % E).astype(jnp.int32)
    lo_f = lo.ravel()[sel].astype(jnp.int32)
    hi_f = hi.ravel()[sel].astype(jnp.int32)
    validf = valid[sel]
    n_real = jnp.sum(valid.astype(jnp.int32))
    last = jnp.maximum(n_real - 1, 0)
    tile_w = jnp.where(validf, tile_f, tile_f[last])
    exp_w = jnp.where(validf, exp_f, exp_f[last])
    lo_w = jnp.where(validf, lo_f, 0)
    hi_w = jnp.where(validf, hi_f, 0)
    return tile_w, exp_w, lo_w, hi_w


def kernel(x, mapped_tokens, Wg, bg, Wi, bi, Wo, bo):
    Bv, Tv, C = x.shape
    xf = x.reshape(Tv * Bv, C)
    mt = mapped_tokens.astype(jnp.int32)

    perm = jnp.argsort(mt, stable=True).astype(jnp.int32)
    inv_perm = jnp.argsort(perm).astype(jnp.int32)
    tile_w, exp_w, lo_w, hi_w = _worklist(mt)

    row_gather = _make_sc_row_gather(T, DIM)
    x_sorted = row_gather(xf, perm)
    out_sorted = _grouped_ffn(tile_w, exp_w, lo_w, hi_w,
                              x_sorted, Wg, bg, Wi, bi, Wo, bo)
    out = row_gather(out_sorted, inv_perm)
    return out.reshape(Bv, Tv, C)


# speedup vs baseline: 3.8298x; 1.0947x over previous
import functools

import jax
import jax.numpy as jnp
from jax import lax
from jax.experimental import pallas as pl
from jax.experimental.pallas import tpu as pltpu
from jax.experimental.pallas import tpu_sc as plsc

DIM = 768
HID = DIM * 4
E = 16
T = 2048
TILE = 128
NT = T // TILE
W = NT + E - 1
MROW = 16
MCOL = T // MROW


def _routing_body(mt_ref, pos_ref, tile_ref, exp_ref, lo_ref, hi_ref, off_ref):
    mt = mt_ref[...]
    acc = jnp.zeros((MROW, MCOL), jnp.int32)
    off = jnp.int32(0)
    off_ref[0] = 0
    for e in range(E):
        m = (mt == e).astype(jnp.int32)
        p = m
        s = 1
        while s < MCOL:
            p = p + jnp.concatenate(
                [jnp.zeros((MROW, s), jnp.int32), p[:, :MCOL - s]], axis=1)
            s *= 2
        rt = p[:, MCOL - 1:MCOL]
        q = rt
        s = 1
        while s < MROW:
            q = q + jnp.concatenate(
                [jnp.zeros((s, 1), jnp.int32), q[:MROW - s, :]], axis=0)
            s *= 2
        rank = p + (q - rt)
        acc = acc + m * (off + rank - 1)
        off = off + jnp.sum(m)
        off_ref[e + 1] = off
    pos_ref[...] = acc

    def merge_step(w, carry):
        i, j = carry
        tb = jnp.where(i < NT, i * TILE, T + 1)
        ob = jnp.where(j < E - 1, off_ref[jnp.minimum(j + 1, E)], T + 1)
        take_e = ob <= tb
        lo_ref[w] = jnp.where(take_e, ob, tb)
        j2 = jnp.where(take_e, j + 1, j)
        exp_ref[w] = j2
        return (jnp.where(take_e, i, i + 1), j2)

    lax.fori_loop(0, W, merge_step, (jnp.int32(0), jnp.int32(0)))

    def fix_step(k, next_exp):
        w = W - 1 - k
        lo = lo_ref[w]
        hi = jnp.where(w == W - 1, T, lo_ref[jnp.minimum(w + 1, W - 1)])
        hi_ref[w] = hi
        tile_ref[w] = jnp.minimum(lo // TILE, NT - 1)
        e2 = jnp.where(lo < hi, exp_ref[w], next_exp)
        exp_ref[w] = e2
        return e2

    lax.fori_loop(0, W, fix_step, exp_ref[W - 1])


def _routing(mt2d):
    smem_i32 = lambda n: jax.ShapeDtypeStruct((n,), jnp.int32)
    return pl.pallas_call(
        _routing_body,
        out_shape=[
            jax.ShapeDtypeStruct((MROW, MCOL), jnp.int32),
            smem_i32(W), smem_i32(W), smem_i32(W), smem_i32(W),
        ],
        out_specs=[
            pl.BlockSpec(memory_space=pltpu.MemorySpace.VMEM),
            pl.BlockSpec(memory_space=pltpu.SMEM),
            pl.BlockSpec(memory_space=pltpu.SMEM),
            pl.BlockSpec(memory_space=pltpu.SMEM),
            pl.BlockSpec(memory_space=pltpu.SMEM),
        ],
        in_specs=[pl.BlockSpec(memory_space=pltpu.MemorySpace.VMEM)],
        scratch_shapes=[pltpu.SMEM((E + 1,), jnp.int32)],
    )(mt2d)


def _make_sc_row_perm(n_rows, dim, scatter):
    info = plsc.get_sparse_core_info()
    nc, ns = info.num_cores, info.num_subcores
    nw = nc * ns
    per_w = n_rows // nw
    mesh = plsc.VectorSubcoreMesh(core_axis_name="c", subcore_axis_name="s")

    @functools.partial(
        pl.kernel,
        mesh=mesh,
        out_type=jax.ShapeDtypeStruct((n_rows, dim), jnp.float32),
        scratch_types=[
            pltpu.VMEM((per_w,), jnp.int32),
            pltpu.VMEM((per_w, dim), jnp.float32),
            pltpu.SemaphoreType.DMA,
        ],
    )
    def perm_k(rows_hbm, idx_hbm, out_hbm, idx_v, rows_v, sem):
        wid = lax.axis_index("s") * nc + lax.axis_index("c")
        base = wid * per_w
        pltpu.sync_copy(idx_hbm.at[pl.ds(base, per_w)], idx_v)
        if scatter:
            pltpu.sync_copy(rows_hbm.at[pl.ds(base, per_w)], rows_v)
            pltpu.async_copy(rows_v, out_hbm.at[idx_v], sem).wait()
        else:
            pltpu.async_copy(rows_hbm.at[idx_v], rows_v, sem).wait()
            pltpu.sync_copy(rows_v, out_hbm.at[pl.ds(base, per_w)])

    return perm_k


def _ffn_body(tid_ref, eid_ref, lo_ref, hi_ref,
              x_ref, wg_ref, bg_ref, wi_ref, bi_ref, wo_ref, bo_ref,
              out_ref):
    w = pl.program_id(0)
    tid = tid_ref[w]
    prev_tid = tid_ref[jnp.maximum(w - 1, 0)]
    first_visit = jnp.logical_or(w == 0, tid != prev_tid)

    @pl.when(first_visit)
    def _init():
        out_ref[...] = jnp.zeros_like(out_ref)

    lo = lo_ref[w]
    hi = hi_ref[w]

    @pl.when(lo < hi)
    def _compute():
        xb = x_ref[...]
        g = jnp.dot(xb, wg_ref[0], preferred_element_type=jnp.float32)
        g = g + bg_ref[0]
        i_ = jnp.dot(xb, wi_ref[0], preferred_element_type=jnp.float32)
        i_ = i_ + bi_ref[0]
        h = (g * jax.nn.sigmoid(g)) * i_
        o = jnp.dot(h, wo_ref[0], preferred_element_type=jnp.float32)
        o = o + bo_ref[0]
        rows = tid * TILE + lax.broadcasted_iota(jnp.int32, (TILE, 1), 0)
        mask = jnp.logical_and(rows >= lo, rows < hi)
        out_ref[...] += jnp.where(mask, o, 0.0)


def _grouped_ffn(tile_w, exp_w, lo_w, hi_w, x_sorted, Wg, bg, Wi, bi, Wo, bo):
    grid_spec = pltpu.PrefetchScalarGridSpec(
        num_scalar_prefetch=4,
        grid=(W,),
        in_specs=[
            pl.BlockSpec((TILE, DIM), lambda w, tid, eid, lo, hi: (tid[w], 0)),
            pl.BlockSpec((1, DIM, HID), lambda w, tid, eid, lo, hi: (eid[w], 0, 0)),
            pl.BlockSpec((1, 1, HID), lambda w, tid, eid, lo, hi: (eid[w], 0, 0)),
            pl.BlockSpec((1, DIM, HID), lambda w, tid, eid, lo, hi: (eid[w], 0, 0)),
            pl.BlockSpec((1, 1, HID), lambda w, tid, eid, lo, hi: (eid[w], 0, 0)),
            pl.BlockSpec((1, HID, DIM), lambda w, tid, eid, lo, hi: (eid[w], 0, 0)),
            pl.BlockSpec((1, 1, DIM), lambda w, tid, eid, lo, hi: (eid[w], 0, 0)),
        ],
        out_specs=pl.BlockSpec((TILE, DIM), lambda w, tid, eid, lo, hi: (tid[w], 0)),
    )
    return pl.pallas_call(
        _ffn_body,
        grid_spec=grid_spec,
        out_shape=jax.ShapeDtypeStruct((T, DIM), jnp.float32),
    )(tile_w, exp_w, lo_w, hi_w, x_sorted,
      Wg, bg.reshape(E, 1, HID), Wi, bi.reshape(E, 1, HID),
      Wo, bo.reshape(E, 1, DIM))


def kernel(x, mapped_tokens, Wg, bg, Wi, bi, Wo, bo):
    Bv, Tv, C = x.shape
    xf = x.reshape(Bv * Tv, C)
    mt2d = mapped_tokens.astype(jnp.int32).reshape(MROW, MCOL)

    pos2d, tile_w, exp_w, lo_w, hi_w = _routing(mt2d)
    pos = pos2d.reshape(T)

    x_sorted = _make_sc_row_perm(T, DIM, scatter=True)(xf, pos)
    out_sorted = _grouped_ffn(tile_w, exp_w, lo_w, hi_w,
                              x_sorted, Wg, bg, Wi, bi, Wo, bo)
    out = _make_sc_row_perm(T, DIM, scatter=False)(out_sorted, pos)
    return out.reshape(Bv, Tv, C)


# speedup vs baseline: 4.6654x; 1.2182x over previous
import functools

import jax
import jax.numpy as jnp
from jax import lax
from jax.experimental import pallas as pl
from jax.experimental.pallas import tpu as pltpu
from jax.experimental.pallas import tpu_sc as plsc

DIM = 768
HID = DIM * 4
E = 16
T = 2048
TILE = 128
ALIGN = 8
PAD = T + E * ALIGN + TILE - ALIGN
PAD = ((PAD + TILE - 1) // TILE) * TILE
NH = 2
HC = HID // NH
MROW = 16
MCOL = T // MROW


def _routing_body(mt_ref, pos_ref, start_ref, cnt_ref):
    mt = mt_ref[...]
    acc = jnp.zeros((MROW, MCOL), jnp.int32)
    ps = jnp.int32(0)
    for e in range(E):
        m = (mt == e).astype(jnp.int32)
        p = m
        s = 1
        while s < MCOL:
            p = p + jnp.concatenate(
                [jnp.zeros((MROW, s), jnp.int32), p[:, :MCOL - s]], axis=1)
            s *= 2
        rt = p[:, MCOL - 1:MCOL]
        q = rt
        s = 1
        while s < MROW:
            q = q + jnp.concatenate(
                [jnp.zeros((s, 1), jnp.int32), q[:MROW - s, :]], axis=0)
            s *= 2
        rank = p + (q - rt)
        acc = acc + m * (ps + rank - 1)
        cnt = jnp.sum(m)
        start_ref[e] = ps
        cnt_ref[e] = cnt
        ps = ps + ((cnt + ALIGN - 1) // ALIGN) * ALIGN
    pos_ref[...] = acc


def _routing(mt2d):
    return pl.pallas_call(
        _routing_body,
        out_shape=[
            jax.ShapeDtypeStruct((MROW, MCOL), jnp.int32),
            jax.ShapeDtypeStruct((E,), jnp.int32),
            jax.ShapeDtypeStruct((E,), jnp.int32),
        ],
        out_specs=[
            pl.BlockSpec(memory_space=pltpu.MemorySpace.VMEM),
            pl.BlockSpec(memory_space=pltpu.SMEM),
            pl.BlockSpec(memory_space=pltpu.SMEM),
        ],
        in_specs=[pl.BlockSpec(memory_space=pltpu.MemorySpace.VMEM)],
    )(mt2d)


def _make_sc_row_perm(n_src, n_dst, dim, scatter):
    info = plsc.get_sparse_core_info()
    nc, ns = info.num_cores, info.num_subcores
    nw = nc * ns
    per_w = n_src // nw
    mesh = plsc.VectorSubcoreMesh(core_axis_name="c", subcore_axis_name="s")

    @functools.partial(
        pl.kernel,
        mesh=mesh,
        out_type=jax.ShapeDtypeStruct((n_dst, dim), jnp.float32),
        scratch_types=[
            pltpu.VMEM((per_w,), jnp.int32),
            pltpu.VMEM((per_w, dim), jnp.float32),
            pltpu.SemaphoreType.DMA,
        ],
    )
    def perm_k(rows_hbm, idx_hbm, out_hbm, idx_v, rows_v, sem):
        wid = lax.axis_index("s") * nc + lax.axis_index("c")
        base = wid * per_w
        pltpu.sync_copy(idx_hbm.at[pl.ds(base, per_w)], idx_v)
        if scatter:
            pltpu.sync_copy(rows_hbm.at[pl.ds(base, per_w)], rows_v)
            pltpu.async_copy(rows_v, out_hbm.at[idx_v], sem).wait()
        else:
            pltpu.async_copy(rows_hbm.at[idx_v], rows_v, sem).wait()
            pltpu.sync_copy(rows_v, out_hbm.at[pl.ds(base, per_w)])

    return perm_k


def _ffn_body(start_ref, cnt_ref,
              x_ref, wg_ref, bg_ref, wi_ref, bi_ref, wo_ref, bo_ref,
              out_ref):
    e = pl.program_id(0)
    h = pl.program_id(1)
    start = start_ref[e]
    cnt = cnt_ref[e]
    ntiles = (cnt + TILE - 1) // TILE
    wg = wg_ref[0]
    wi = wi_ref[0]
    wo = wo_ref[0]
    bg = bg_ref[0, 0]
    bi = bi_ref[0, 0]
    bo = bo_ref[0, 0]

    def tile_body(k, _):
        base = pl.multiple_of(start, ALIGN) + k * TILE
        xb = x_ref[pl.ds(base, TILE), :]
        g = jnp.dot(xb, wg, preferred_element_type=jnp.float32) + bg
        i_ = jnp.dot(xb, wi, preferred_element_type=jnp.float32) + bi
        hdn = (g * jax.nn.sigmoid(g)) * i_
        o = jnp.dot(hdn, wo, preferred_element_type=jnp.float32)
        rows = k * TILE + lax.broadcasted_iota(jnp.int32, (TILE, 1), 0)
        mask = rows < cnt
        bo_term = jnp.where(h == 0, bo, 0.0)
        prev = jnp.where(h == 0, 0.0, out_ref[pl.ds(base, TILE), :])
        out_ref[pl.ds(base, TILE), :] = prev + jnp.where(mask, o + bo_term, 0.0)
        return 0

    lax.fori_loop(0, ntiles, tile_body, 0)


def _grouped_ffn(start_w, cnt_w, x_pad, Wg, bg, Wi, bi, Wo, bo):
    grid_spec = pltpu.PrefetchScalarGridSpec(
        num_scalar_prefetch=2,
        grid=(E, NH),
        in_specs=[
            pl.BlockSpec((PAD, DIM), lambda e, h, st, cn: (0, 0)),
            pl.BlockSpec((1, DIM, HC), lambda e, h, st, cn: (e, 0, h)),
            pl.BlockSpec((1, 1, HC), lambda e, h, st, cn: (e, 0, h)),
            pl.BlockSpec((1, DIM, HC), lambda e, h, st, cn: (e, 0, h)),
            pl.BlockSpec((1, 1, HC), lambda e, h, st, cn: (e, 0, h)),
            pl.BlockSpec((1, HC, DIM), lambda e, h, st, cn: (e, h, 0)),
            pl.BlockSpec((1, 1, DIM), lambda e, h, st, cn: (e, 0, 0)),
        ],
        out_specs=pl.BlockSpec((PAD, DIM), lambda e, h, st, cn: (0, 0)),
    )
    return pl.pallas_call(
        _ffn_body,
        grid_spec=grid_spec,
        out_shape=jax.ShapeDtypeStruct((PAD, DIM), jnp.float32),
    )(start_w, cnt_w, x_pad,
      Wg, bg.reshape(E, 1, HID), Wi, bi.reshape(E, 1, HID),
      Wo, bo.reshape(E, 1, DIM))


def kernel(x, mapped_tokens, Wg, bg, Wi, bi, Wo, bo):
    Bv, Tv, C = x.shape
    xf = x.reshape(Bv * Tv, C)
    mt2d = mapped_tokens.astype(jnp.int32).reshape(MROW, MCOL)

    pos2d, start_w, cnt_w = _routing(mt2d)
    pos = pos2d.reshape(T)

    x_pad = _make_sc_row_perm(T, PAD, DIM, scatter=True)(xf, pos)
    out_pad = _grouped_ffn(start_w, cnt_w, x_pad, Wg, bg, Wi, bi, Wo, bo)
    out = _make_sc_row_perm(T, T, DIM, scatter=False)(out_pad, pos)
    return out.reshape(Bv, Tv, C)


# speedup vs baseline: 4.6713x; 1.0013x over previous
import functools

import jax
import jax.numpy as jnp
from jax import lax
from jax.experimental import pallas as pl
from jax.experimental.pallas import tpu as pltpu
from jax.experimental.pallas import tpu_sc as plsc

DIM = 768
HID = DIM * 4
E = 16
T = 2048
TILE = 256
ALIGN = 8
PAD = T + E * ALIGN + TILE - ALIGN
PAD = ((PAD + TILE - 1) // TILE) * TILE
NH = 2
HC = HID // NH
MROW = 16
MCOL = T // MROW


def _routing_body(mt_ref, pos_ref, start_ref, cnt_ref):
    mt = mt_ref[...]
    acc = jnp.zeros((MROW, MCOL), jnp.int32)
    ps = jnp.int32(0)
    for e in range(E):
        m = (mt == e).astype(jnp.int32)
        p = m
        s = 1
        while s < MCOL:
            p = p + jnp.concatenate(
                [jnp.zeros((MROW, s), jnp.int32), p[:, :MCOL - s]], axis=1)
            s *= 2
        rt = p[:, MCOL - 1:MCOL]
        q = rt
        s = 1
        while s < MROW:
            q = q + jnp.concatenate(
                [jnp.zeros((s, 1), jnp.int32), q[:MROW - s, :]], axis=0)
            s *= 2
        rank = p + (q - rt)
        acc = acc + m * (ps + rank - 1)
        cnt = jnp.sum(m)
        start_ref[e] = ps
        cnt_ref[e] = cnt
        ps = ps + ((cnt + ALIGN - 1) // ALIGN) * ALIGN
    pos_ref[...] = acc


def _routing(mt2d):
    return pl.pallas_call(
        _routing_body,
        out_shape=[
            jax.ShapeDtypeStruct((MROW, MCOL), jnp.int32),
            jax.ShapeDtypeStruct((E,), jnp.int32),
            jax.ShapeDtypeStruct((E,), jnp.int32),
        ],
        out_specs=[
            pl.BlockSpec(memory_space=pltpu.MemorySpace.VMEM),
            pl.BlockSpec(memory_space=pltpu.SMEM),
            pl.BlockSpec(memory_space=pltpu.SMEM),
        ],
        in_specs=[pl.BlockSpec(memory_space=pltpu.MemorySpace.VMEM)],
    )(mt2d)


def _make_sc_row_perm(n_src, n_dst, dim, scatter):
    info = plsc.get_sparse_core_info()
    nc, ns = info.num_cores, info.num_subcores
    nw = nc * ns
    per_w = n_src // nw
    mesh = plsc.VectorSubcoreMesh(core_axis_name="c", subcore_axis_name="s")

    @functools.partial(
        pl.kernel,
        mesh=mesh,
        out_type=jax.ShapeDtypeStruct((n_dst, dim), jnp.float32),
        scratch_types=[
            pltpu.VMEM((per_w,), jnp.int32),
            pltpu.VMEM((per_w, dim), jnp.float32),
            pltpu.SemaphoreType.DMA,
        ],
    )
    def perm_k(rows_hbm, idx_hbm, out_hbm, idx_v, rows_v, sem):
        wid = lax.axis_index("s") * nc + lax.axis_index("c")
        base = wid * per_w
        pltpu.sync_copy(idx_hbm.at[pl.ds(base, per_w)], idx_v)
        if scatter:
            pltpu.sync_copy(rows_hbm.at[pl.ds(base, per_w)], rows_v)
            pltpu.async_copy(rows_v, out_hbm.at[idx_v], sem).wait()
        else:
            pltpu.async_copy(rows_hbm.at[idx_v], rows_v, sem).wait()
            pltpu.sync_copy(rows_v, out_hbm.at[pl.ds(base, per_w)])

    return perm_k


def _ffn_body(start_ref, cnt_ref,
              x_ref, wg_ref, bg_ref, wi_ref, bi_ref, wo_ref, bo_ref,
              out_ref):
    e = pl.program_id(0)
    h = pl.program_id(1)
    start = start_ref[e]
    cnt = cnt_ref[e]
    ntiles = (cnt + TILE - 1) // TILE
    wg = wg_ref[0]
    wi = wi_ref[0]
    wo = wo_ref[0]
    bg = bg_ref[0, 0]
    bi = bi_ref[0, 0]
    bo = bo_ref[0, 0]

    def tile_body(k, _):
        base = pl.multiple_of(start, ALIGN) + k * TILE
        xb = x_ref[pl.ds(base, TILE), :]
        g = jnp.dot(xb, wg, preferred_element_type=jnp.float32) + bg
        i_ = jnp.dot(xb, wi, preferred_element_type=jnp.float32) + bi
        hdn = (g * jax.nn.sigmoid(g)) * i_
        o = jnp.dot(hdn, wo, preferred_element_type=jnp.float32)
        rows = k * TILE + lax.broadcasted_iota(jnp.int32, (TILE, 1), 0)
        mask = rows < cnt
        bo_term = jnp.where(h == 0, bo, 0.0)
        prev = jnp.where(h == 0, 0.0, out_ref[pl.ds(base, TILE), :])
        out_ref[pl.ds(base, TILE), :] = prev + jnp.where(mask, o + bo_term, 0.0)
        return 0

    lax.fori_loop(0, ntiles, tile_body, 0)


def _grouped_ffn(start_w, cnt_w, x_pad, Wg, bg, Wi, bi, Wo, bo):
    grid_spec = pltpu.PrefetchScalarGridSpec(
        num_scalar_prefetch=2,
        grid=(E, NH),
        in_specs=[
            pl.BlockSpec((PAD, DIM), lambda e, h, st, cn: (0, 0)),
            pl.BlockSpec((1, DIM, HC), lambda e, h, st, cn: (e, 0, h)),
            pl.BlockSpec((1, 1, HC), lambda e, h, st, cn: (e, 0, h)),
            pl.BlockSpec((1, DIM, HC), lambda e, h, st, cn: (e, 0, h)),
            pl.BlockSpec((1, 1, HC), lambda e, h, st, cn: (e, 0, h)),
            pl.BlockSpec((1, HC, DIM), lambda e, h, st, cn: (e, h, 0)),
            pl.BlockSpec((1, 1, DIM), lambda e, h, st, cn: (e, 0, 0)),
        ],
        out_specs=pl.BlockSpec((PAD, DIM), lambda e, h, st, cn: (0, 0)),
    )
    return pl.pallas_call(
        _ffn_body,
        grid_spec=grid_spec,
        out_shape=jax.ShapeDtypeStruct((PAD, DIM), jnp.float32),
    )(start_w, cnt_w, x_pad,
      Wg, bg.reshape(E, 1, HID), Wi, bi.reshape(E, 1, HID),
      Wo, bo.reshape(E, 1, DIM))


def kernel(x, mapped_tokens, Wg, bg, Wi, bi, Wo, bo):
    Bv, Tv, C = x.shape
    xf = x.reshape(Bv * Tv, C)
    mt2d = mapped_tokens.astype(jnp.int32).reshape(MROW, MCOL)

    pos2d, start_w, cnt_w = _routing(mt2d)
    pos = pos2d.reshape(T)

    x_pad = _make_sc_row_perm(T, PAD, DIM, scatter=True)(xf, pos)
    out_pad = _grouped_ffn(start_w, cnt_w, x_pad, Wg, bg, Wi, bi, Wo, bo)
    out = _make_sc_row_perm(T, T, DIM, scatter=False)(out_pad, pos)
    return out.reshape(Bv, Tv, C)
